# baseline (speedup 1.0000x reference)
import jax
import jax.numpy as jnp
from jax import lax
from jax.experimental import pallas as pl
from jax.experimental.pallas import tpu as pltpu
from jax.experimental.pallas import tpu_sc as plsc

_N = 10000
_E = 320000
_MSG = 16
_HID = 16
_NT = 16

_NC, _NS = 2, 16
_NW = _NC * _NS
_EW = _E // _NW
_C = 400
_NCHUNK = _EW // _C
_RSTRIDE = 624
_RCOPY = 640


def _ytab_body(x_ref, w_ref, y_ref):
    y_ref[...] = jnp.dot(x_ref[...], w_ref[...],
                         preferred_element_type=jnp.float32)


def _ytab(x, w):
    rb = 2000
    return pl.pallas_call(
        _ytab_body,
        grid=(_N // rb,),
        in_specs=[pl.BlockSpec((rb, _HID), lambda i: (i, 0)),
                  pl.BlockSpec((_HID, _NT * _MSG), lambda i: (0, 0))],
        out_specs=pl.BlockSpec((rb, _NT * _MSG), lambda i: (i, 0)),
        out_shape=jax.ShapeDtypeStruct((_N, _NT * _MSG), jnp.float32),
    )(x, w)


_NBUF = 5


def _sc_body(y_hbm, ei_hbm, et_hbm, out_hbm,
             sfull, dfull, gfull, zbuf, rows, acc, gsems, ssems):
    c = lax.axis_index("c")
    s = lax.axis_index("s")
    wid = s * _NC + c
    base = wid * _EW
    row0 = s * _RSTRIDE

    cp_s = pltpu.async_copy(ei_hbm.at[0, pl.ds(base, _EW)], sfull, gsems[0])
    cp_t = pltpu.async_copy(et_hbm.at[pl.ds(base, _EW)], gfull, gsems[1])
    cp_d = pltpu.async_copy(ei_hbm.at[1, pl.ds(base, _EW)], dfull, gsems[2])

    zeros16 = jnp.zeros((16,), jnp.float32)

    def zrow(r, carry):
        for u in range(8):
            zbuf[r * 8 + u, :] = zeros16
        return carry
    lax.fori_loop(0, _RCOPY // 8, zrow, 0)
    cp_z = pltpu.async_copy(zbuf, acc.at[pl.ds(row0, _RCOPY)], ssems[0])

    cp_s.wait()
    cp_t.wait()

    def gidx(k, carry):
        for u in range(5):
            sl = pl.ds((k * 5 + u) * 16, 16)
            gfull[sl] = sfull[sl] * _NT + gfull[sl]
        return carry
    lax.fori_loop(0, _EW // 80, gidx, 0)
    cp_d.wait()
    cp_z.wait()

    plsc.subcore_barrier()

    def g_issue(j, b):
        pltpu.async_copy(y_hbm.at[gfull.at[pl.ds(j * _C, _C)]],
                         rows[b], gsems[b])

    def g_wait(b):
        pltpu.make_async_copy(y_hbm.at[gfull.at[pl.ds(0, _C)]],
                              rows[b], gsems[b]).wait()

    def s_issue(j, b):
        pltpu.async_copy(rows[b], acc.at[dfull.at[pl.ds(j * _C, _C)]],
                         ssems[b], add=True)

    def s_wait(b):
        pltpu.make_async_copy(rows[b], acc.at[dfull.at[pl.ds(0, _C)]],
                              ssems[b]).wait()

    for b in range(_NBUF):
        g_issue(b, b)

    def body(jj, carry):
        j = jj * _NBUF
        for b in range(_NBUF):
            g_wait(b)
            s_issue(j + b, b)
        for b in range(_NBUF):
            s_wait(b)

            @pl.when(j + _NBUF + b < _NCHUNK)
            def _():
                g_issue(j + _NBUF + b, b)
        return carry

    lax.fori_loop(0, _NCHUNK // _NBUF, body, 0)
    plsc.subcore_barrier()
    pltpu.sync_copy(acc.at[pl.ds(row0, _RCOPY)],
                    out_hbm.at[c, pl.ds(row0, _RCOPY)])


def _sc_partials(y, ei, et):
    f = pl.kernel(
        _sc_body,
        out_type=jax.ShapeDtypeStruct((_NC, _N, _MSG), jnp.float32),
        mesh=plsc.VectorSubcoreMesh(core_axis_name="c", subcore_axis_name="s"),
        scratch_types=[
            pltpu.VMEM((_EW,), jnp.int32),
            pltpu.VMEM((_EW,), jnp.int32),
            pltpu.VMEM((_EW,), jnp.int32),
            pltpu.VMEM((_RCOPY, _MSG), jnp.float32),
            [pltpu.VMEM((_C, _MSG), jnp.float32) for _ in range(_NBUF)],
            pltpu.VMEM_SHARED((_N, _MSG), jnp.float32),
            [pltpu.SemaphoreType.DMA for _ in range(_NBUF)],
            [pltpu.SemaphoreType.DMA for _ in range(_NBUF)],
        ],
        compiler_params=pltpu.CompilerParams(use_tc_tiling_on_sc=False),
    )
    return f(y, ei, et)


def _gru_body(pp_hbm, x_ref, wih_ref, whh_ref, bih_ref, bhh_ref, o_ref,
              pp_ref, sem):
    i = pl.program_id(0)
    rb = o_ref.shape[0]
    pltpu.make_async_copy(pp_hbm.at[:, pl.ds(i * rb, rb), :], pp_ref,
                          sem).start()
    pltpu.make_async_copy(pp_hbm.at[:, pl.ds(i * rb, rb), :], pp_ref,
                          sem).wait()
    m = pp_ref[0] + pp_ref[1]
    h = x_ref[...]
    dn = (((1,), (1,)), ((), ()))
    gi = lax.dot_general(wih_ref[...], m, dn,
                         preferred_element_type=jnp.float32) + bih_ref[...]
    gh = lax.dot_general(whh_ref[...], h, dn,
                         preferred_element_type=jnp.float32) + bhh_ref[...]
    ht = h.T
    r = jax.nn.sigmoid(gi[:_HID] + gh[:_HID])
    z = jax.nn.sigmoid(gi[_HID:2 * _HID] + gh[_HID:2 * _HID])
    n = jnp.tanh(gi[2 * _HID:] + r * gh[2 * _HID:])
    o_ref[...] = ((1.0 - z) * n + z * ht).T


def _gru(partials, x, w_ih, w_hh, b_ih, b_hh):
    rb = 2000
    g3 = 3 * _HID
    return pl.pallas_call(
        _gru_body,
        grid=(_N // rb,),
        in_specs=[pl.BlockSpec(memory_space=pl.ANY),
                  pl.BlockSpec((rb, _HID), lambda i: (i, 0)),
                  pl.BlockSpec((g3, _MSG), lambda i: (0, 0)),
                  pl.BlockSpec((g3, _HID), lambda i: (0, 0)),
                  pl.BlockSpec((g3, 1), lambda i: (0, 0)),
                  pl.BlockSpec((g3, 1), lambda i: (0, 0))],
        out_specs=pl.BlockSpec((rb, _HID), lambda i: (i, 0)),
        out_shape=jax.ShapeDtypeStruct((_N, _HID), jnp.float32),
        scratch_shapes=[pltpu.VMEM((_NC, rb, _MSG), jnp.float32),
                        pltpu.SemaphoreType.DMA],
    )(partials, x, w_ih, w_hh, b_ih, b_hh)


def kernel(x, edge_index, edge_type, edge_matrix, W_ih, W_hh, b_ih, b_hh):
    w = edge_matrix.reshape(_NT, _MSG, _HID).transpose(2, 0, 1)
    w = w.reshape(_HID, _NT * _MSG)
    y = _ytab(x, w)
    partials = _sc_partials(y.reshape(_N * _NT, _MSG), edge_index, edge_type)
    return _gru(partials, x, W_ih, W_hh,
                b_ih.reshape(-1, 1), b_hh.reshape(-1, 1))


# speedup vs baseline: 1.0816x; 1.0816x over previous
import jax
import jax.numpy as jnp
from jax import lax
from jax.experimental import pallas as pl
from jax.experimental.pallas import tpu as pltpu
from jax.experimental.pallas import tpu_sc as plsc

_N = 10000
_E = 320000
_MSG = 16
_HID = 16
_NT = 16

_NC, _NS = 2, 16
_NW = _NC * _NS
_EW = _E // _NW
_C = 400
_NCHUNK = _EW // _C
_RSTRIDE = 624
_RCOPY = 640


def _ytab_body(x_ref, w_ref, y_ref):
    y_ref[...] = jnp.dot(x_ref[...], w_ref[...],
                         preferred_element_type=jnp.float32)


def _ytab(x, w):
    rb = 2000
    return pl.pallas_call(
        _ytab_body,
        grid=(_N // rb,),
        in_specs=[pl.BlockSpec((rb, _HID), lambda i: (i, 0)),
                  pl.BlockSpec((_HID, _NT * _MSG), lambda i: (0, 0))],
        out_specs=pl.BlockSpec((rb, _NT * _MSG), lambda i: (i, 0)),
        out_shape=jax.ShapeDtypeStruct((_N, _NT * _MSG), jnp.float32),
    )(x, w)


_NBUF = 5


def _sc_body(y_hbm, ei_hbm, et_hbm, out_hbm,
             sfull, dfull, gfull, zbuf, rows, acc, gsems, ssems):
    c = lax.axis_index("c")
    s = lax.axis_index("s")
    wid = s * _NC + c
    base = wid * _EW
    row0 = s * _RSTRIDE

    cp_s = pltpu.async_copy(ei_hbm.at[0, pl.ds(base, _EW)], sfull, gsems[0])
    cp_t = pltpu.async_copy(et_hbm.at[pl.ds(base, _EW)], gfull, gsems[1])
    cp_d = pltpu.async_copy(ei_hbm.at[1, pl.ds(base, _EW)], dfull, gsems[2])

    zeros16 = jnp.zeros((16,), jnp.float32)

    def zrow(r, carry):
        for u in range(8):
            zbuf[r * 8 + u, :] = zeros16
        return carry
    lax.fori_loop(0, _RCOPY // 8, zrow, 0)
    cp_z = pltpu.async_copy(zbuf, acc.at[pl.ds(row0, _RCOPY)], ssems[0])

    cp_s.wait()
    cp_t.wait()

    def gidx(k, carry):
        for u in range(5):
            sl = pl.ds((k * 5 + u) * 16, 16)
            gfull[sl] = sfull[sl] * _NT + gfull[sl]
        return carry
    lax.fori_loop(0, _EW // 80, gidx, 0)
    cp_d.wait()
    cp_z.wait()

    plsc.subcore_barrier()

    def g_issue(j, b):
        pltpu.async_copy(y_hbm.at[gfull.at[pl.ds(j * _C, _C)]],
                         rows[b], gsems[b])

    def g_wait(b):
        pltpu.make_async_copy(y_hbm.at[gfull.at[pl.ds(0, _C)]],
                              rows[b], gsems[b]).wait()

    def s_issue(j, b):
        pltpu.async_copy(rows[b], acc.at[dfull.at[pl.ds(j * _C, _C)]],
                         ssems[b], add=True)

    def s_wait(b):
        pltpu.make_async_copy(rows[b], acc.at[dfull.at[pl.ds(0, _C)]],
                              ssems[b]).wait()

    for b in range(_NBUF):
        g_issue(b, b)

    def body(jj, carry):
        j = jj * _NBUF
        for b in range(_NBUF):
            g_wait(b)
            s_issue(j + b, b)
        for b in range(_NBUF):
            s_wait(b)

            @pl.when(j + _NBUF + b < _NCHUNK)
            def _():
                g_issue(j + _NBUF + b, b)
        return carry

    lax.fori_loop(0, _NCHUNK // _NBUF, body, 0)
    plsc.subcore_barrier()
    pltpu.sync_copy(acc.at[pl.ds(row0, _RCOPY)],
                    out_hbm.at[c, pl.ds(row0, _RCOPY)])


def _sc_partials(y, ei, et):
    f = pl.kernel(
        _sc_body,
        out_type=jax.ShapeDtypeStruct((_NC, _N, _MSG), jnp.float32),
        mesh=plsc.VectorSubcoreMesh(core_axis_name="c", subcore_axis_name="s"),
        scratch_types=[
            pltpu.VMEM((_EW,), jnp.int32),
            pltpu.VMEM((_EW,), jnp.int32),
            pltpu.VMEM((_EW,), jnp.int32),
            pltpu.VMEM((_RCOPY, _MSG), jnp.float32),
            [pltpu.VMEM((_C, _MSG), jnp.float32) for _ in range(_NBUF)],
            pltpu.VMEM_SHARED((_N, _MSG), jnp.float32),
            [pltpu.SemaphoreType.DMA for _ in range(_NBUF)],
            [pltpu.SemaphoreType.DMA for _ in range(_NBUF)],
        ],
        compiler_params=pltpu.CompilerParams(use_tc_tiling_on_sc=False),
    )
    return f(y, ei, et)


def _gru_body(pp_ref, x_ref, wih_ref, whh_ref, bih_ref, bhh_ref, o_ref):
    m = pp_ref[0] + pp_ref[1]
    h = x_ref[...]
    dn = (((1,), (1,)), ((), ()))
    gi = lax.dot_general(wih_ref[...], m, dn,
                         preferred_element_type=jnp.float32) + bih_ref[...]
    gh = lax.dot_general(whh_ref[...], h, dn,
                         preferred_element_type=jnp.float32) + bhh_ref[...]
    ht = h.T
    r = jax.nn.sigmoid(gi[:_HID] + gh[:_HID])
    z = jax.nn.sigmoid(gi[_HID:2 * _HID] + gh[_HID:2 * _HID])
    n = jnp.tanh(gi[2 * _HID:] + r * gh[2 * _HID:])
    o_ref[...] = ((1.0 - z) * n + z * ht).T


def _gru(partials, x, w_ih, w_hh, b_ih, b_hh):
    rb = 2000
    g3 = 3 * _HID
    return pl.pallas_call(
        _gru_body,
        grid=(_N // rb,),
        in_specs=[pl.BlockSpec((_NC, rb, _MSG), lambda i: (0, i, 0)),
                  pl.BlockSpec((rb, _HID), lambda i: (i, 0)),
                  pl.BlockSpec((g3, _MSG), lambda i: (0, 0)),
                  pl.BlockSpec((g3, _HID), lambda i: (0, 0)),
                  pl.BlockSpec((g3, 1), lambda i: (0, 0)),
                  pl.BlockSpec((g3, 1), lambda i: (0, 0))],
        out_specs=pl.BlockSpec((rb, _HID), lambda i: (i, 0)),
        out_shape=jax.ShapeDtypeStruct((_N, _HID), jnp.float32),
    )(partials, x, w_ih, w_hh, b_ih, b_hh)


def kernel(x, edge_index, edge_type, edge_matrix, W_ih, W_hh, b_ih, b_hh):
    w = edge_matrix.reshape(_NT, _MSG, _HID).transpose(2, 0, 1)
    w = w.reshape(_HID, _NT * _MSG)
    y = _ytab(x, w)
    partials = _sc_partials(y.reshape(_N * _NT, _MSG), edge_index, edge_type)
    return _gru(partials, x, W_ih, W_hh,
                b_ih.reshape(-1, 1), b_hh.reshape(-1, 1))
